# baseline (speedup 1.0000x reference)
import functools

import jax
import jax.numpy as jnp
from jax import lax
from jax.experimental import pallas as pl
from jax.experimental.pallas import tpu as pltpu
from jax.experimental.pallas import tpu_sc as plsc

B, L, F = 16, 2048, 512
ROWS = B * L

_LBLK = 128


def _mask_body(x_ref, o_ref):
    nz = (x_ref[...] != 0.0).astype(jnp.float32)
    s = jnp.sum(jnp.sum(nz, axis=2), axis=0, keepdims=True)
    o_ref[...] = (s > 0.0).astype(jnp.int32)


_colmask_call = pl.pallas_call(
    _mask_body,
    grid=(L // _LBLK,),
    in_specs=[pl.BlockSpec((B, _LBLK, F), lambda i: (0, i, 0))],
    out_specs=pl.BlockSpec((1, _LBLK), lambda i: (0, i)),
    out_shape=jax.ShapeDtypeStruct((1, L), jnp.int32),
)


def _dest_body(m_ref, o_ref):
    kept = m_ref[...]
    r = lax.broadcasted_iota(jnp.int32, (L, L), 0)
    c = lax.broadcasted_iota(jnp.int32, (L, L), 1)
    tri = (r <= c).astype(jnp.bfloat16)
    incl = lax.dot_general(
        kept.astype(jnp.bfloat16), tri,
        (((1,), (0,)), ((), ())),
        preferred_element_type=jnp.float32,
    ).astype(jnp.int32)
    total = jnp.sum(kept)
    pe = incl - kept
    col = lax.broadcasted_iota(jnp.int32, (1, L), 1)
    dest = jnp.where(kept > 0, pe, total + col - pe)
    row = lax.broadcasted_iota(jnp.int32, (B, L), 0)
    o_ref[...] = dest + row * L


_dest_call = pl.pallas_call(
    _dest_body,
    out_shape=jax.ShapeDtypeStruct((B, L), jnp.int32),
)


_CHUNK = 64


@functools.cache
def _make_scatter():
    info = plsc.get_sparse_core_info()
    nc, ns = info.num_cores, info.num_subcores
    nw = nc * ns
    rpw = ROWS // nw
    nchunks = rpw // _CHUNK
    mesh = plsc.VectorSubcoreMesh(core_axis_name="c", subcore_axis_name="s")

    @functools.partial(
        pl.kernel,
        mesh=mesh,
        out_type=jax.ShapeDtypeStruct((ROWS, F), jnp.float32),
        scratch_types=[
            pltpu.VMEM((nchunks, _CHUNK), jnp.int32),
            pltpu.VMEM((_CHUNK, F), jnp.float32),
            pltpu.VMEM((_CHUNK, F), jnp.float32),
            pltpu.SemaphoreType.DMA,
            pltpu.SemaphoreType.DMA,
        ],
    )
    def scatter(rows_hbm, idx_hbm, out_hbm, idx_v, rows_a, rows_b, sem_a,
                sem_b):
        wid = lax.axis_index("s") * nc + lax.axis_index("c")
        base = wid * rpw
        pltpu.sync_copy(idx_hbm.at[pl.ds(wid * nchunks, nchunks)], idx_v)

        bufs = (rows_a, rows_b)
        sems = (sem_a, sem_b)
        pending = [None, None]
        for j in range(nchunks):
            b = j & 1
            if pending[b] is not None:
                pending[b].wait()
            pltpu.sync_copy(rows_hbm.at[pl.ds(base + j * _CHUNK, _CHUNK)],
                            bufs[b])
            pending[b] = pltpu.async_copy(bufs[b], out_hbm.at[idx_v.at[j]],
                                          sems[b])
        pending[0].wait()
        pending[1].wait()

    return scatter


def _tc_copy_body(x_ref, o_ref):
    o_ref[...] = x_ref[...]


_tc_copy = pl.pallas_call(
    _tc_copy_body,
    grid=(16,),
    in_specs=[pl.BlockSpec((B, L // 16, F), lambda i: (0, i, 0))],
    out_specs=pl.BlockSpec((B, L // 16, F), lambda i: (0, i, 0)),
    out_shape=jax.ShapeDtypeStruct((B, L, F), jnp.float32),
)


@functools.cache
def _make_sc_sample():
    info = plsc.get_sparse_core_info()
    nc, ns = info.num_cores, info.num_subcores
    nw = nc * ns
    cpw = L // nw
    mesh = plsc.VectorSubcoreMesh(core_axis_name="c", subcore_axis_name="s")

    @functools.partial(
        pl.kernel,
        mesh=mesh,
        out_type=jax.ShapeDtypeStruct((nw * cpw * 16,), jnp.int32),
        scratch_types=[
            pltpu.VMEM((cpw * F,), jnp.int32),
            pltpu.VMEM((cpw * 16,), jnp.int32),
        ],
    )
    def sample(flat_hbm, ok_hbm, smp_v, flag_v):
        wid = lax.axis_index("s") * nc + lax.axis_index("c")
        pltpu.sync_copy(flat_hbm.at[pl.ds(wid * cpw * F, cpw * F)], smp_v)
        for c in range(cpw):
            acc = jnp.zeros((16,), jnp.int32)
            for k in range(8):
                v = smp_v[pl.ds(c * F + 16 * k, 16)]
                acc = acc | (v & jnp.int32(0x7FFFFFFF))
            flag_v[pl.ds(c * 16, 16)] = acc
        pltpu.sync_copy(flag_v, ok_hbm.at[pl.ds(wid * cpw * 16, cpw * 16)])

    return sample


def _slow_path(x):
    colmask = _colmask_call(x)
    dest = _dest_call(colmask)
    out = _make_scatter()(x.reshape(ROWS, F),
                          dest.reshape(ROWS // _CHUNK, _CHUNK))
    return out.reshape(B, L, F)


def kernel(inputs):
    copied = _tc_copy(inputs)
    bits = lax.bitcast_convert_type(inputs, jnp.int32).reshape(ROWS * F)
    flags = _make_sc_sample()(bits)
    ok = jnp.all(jnp.any(flags.reshape(L, 16) != 0, axis=1))
    return lax.cond(ok, lambda c, x: c, lambda c, x: _slow_path(x),
                    copied, inputs)


# speedup vs baseline: 1.3496x; 1.3496x over previous
import functools

import jax
import jax.numpy as jnp
from jax import lax
from jax.experimental import pallas as pl
from jax.experimental.pallas import tpu as pltpu
from jax.experimental.pallas import tpu_sc as plsc

B, L, F = 16, 2048, 512
ROWS = B * L

_LBLK = 128


def _mask_body(x_ref, o_ref):
    nz = (x_ref[...] != 0.0).astype(jnp.float32)
    s = jnp.sum(jnp.sum(nz, axis=2), axis=0, keepdims=True)
    o_ref[...] = (s > 0.0).astype(jnp.int32)


_colmask_call = pl.pallas_call(
    _mask_body,
    grid=(L // _LBLK,),
    in_specs=[pl.BlockSpec((B, _LBLK, F), lambda i: (0, i, 0))],
    out_specs=pl.BlockSpec((1, _LBLK), lambda i: (0, i)),
    out_shape=jax.ShapeDtypeStruct((1, L), jnp.int32),
)


def _dest_body(m_ref, o_ref):
    kept = m_ref[...]
    r = lax.broadcasted_iota(jnp.int32, (L, L), 0)
    c = lax.broadcasted_iota(jnp.int32, (L, L), 1)
    tri = (r <= c).astype(jnp.bfloat16)
    incl = lax.dot_general(
        kept.astype(jnp.bfloat16), tri,
        (((1,), (0,)), ((), ())),
        preferred_element_type=jnp.float32,
    ).astype(jnp.int32)
    total = jnp.sum(kept)
    pe = incl - kept
    col = lax.broadcasted_iota(jnp.int32, (1, L), 1)
    dest = jnp.where(kept > 0, pe, total + col - pe)
    row = lax.broadcasted_iota(jnp.int32, (B, L), 0)
    o_ref[...] = dest + row * L


_dest_call = pl.pallas_call(
    _dest_body,
    out_shape=jax.ShapeDtypeStruct((B, L), jnp.int32),
)


_CHUNK = 64


@functools.cache
def _make_scatter():
    info = plsc.get_sparse_core_info()
    nc, ns = info.num_cores, info.num_subcores
    nw = nc * ns
    rpw = ROWS // nw
    nchunks = rpw // _CHUNK
    mesh = plsc.VectorSubcoreMesh(core_axis_name="c", subcore_axis_name="s")

    @functools.partial(
        pl.kernel,
        mesh=mesh,
        out_type=jax.ShapeDtypeStruct((ROWS, F), jnp.float32),
        scratch_types=[
            pltpu.VMEM((nchunks, _CHUNK), jnp.int32),
            pltpu.VMEM((_CHUNK, F), jnp.float32),
            pltpu.VMEM((_CHUNK, F), jnp.float32),
            pltpu.SemaphoreType.DMA,
            pltpu.SemaphoreType.DMA,
        ],
    )
    def scatter(rows_hbm, idx_hbm, out_hbm, idx_v, rows_a, rows_b, sem_a,
                sem_b):
        wid = lax.axis_index("s") * nc + lax.axis_index("c")
        base = wid * rpw
        pltpu.sync_copy(idx_hbm.at[pl.ds(wid * nchunks, nchunks)], idx_v)

        bufs = (rows_a, rows_b)
        sems = (sem_a, sem_b)
        pending = [None, None]
        for j in range(nchunks):
            b = j & 1
            if pending[b] is not None:
                pending[b].wait()
            pltpu.sync_copy(rows_hbm.at[pl.ds(base + j * _CHUNK, _CHUNK)],
                            bufs[b])
            pending[b] = pltpu.async_copy(bufs[b], out_hbm.at[idx_v.at[j]],
                                          sems[b])
        pending[0].wait()
        pending[1].wait()

    return scatter


def _tc_copy_body(x_ref, o_ref):
    o_ref[...] = x_ref[...]


_tc_copy = pl.pallas_call(
    _tc_copy_body,
    grid=(16,),
    in_specs=[pl.BlockSpec((B, L // 16, F), lambda i: (0, i, 0))],
    out_specs=pl.BlockSpec((B, L // 16, F), lambda i: (0, i, 0)),
    out_shape=jax.ShapeDtypeStruct((B, L, F), jnp.float32),
)


@functools.cache
def _make_sc_sample():
    info = plsc.get_sparse_core_info()
    nc, ns = info.num_cores, info.num_subcores
    nw = nc * ns
    cpw = L // nw
    mesh = plsc.VectorSubcoreMesh(core_axis_name="c", subcore_axis_name="s")

    @functools.partial(
        pl.kernel,
        mesh=mesh,
        out_type=jax.ShapeDtypeStruct((nw * cpw * 16,), jnp.float32),
        scratch_types=[
            pltpu.VMEM((cpw * F,), jnp.float32),
            pltpu.VMEM((cpw * 16,), jnp.float32),
        ],
    )
    def sample(flat_hbm, ok_hbm, smp_v, flag_v):
        wid = lax.axis_index("s") * nc + lax.axis_index("c")
        pltpu.sync_copy(flat_hbm.at[pl.ds(wid * cpw * F, cpw * F)], smp_v)
        for c in range(cpw):
            acc = jnp.zeros((16,), jnp.float32)
            for k in range(8):
                acc = acc + jnp.abs(smp_v[pl.ds(c * F + 16 * k, 16)])
            flag_v[pl.ds(c * 16, 16)] = acc
        pltpu.sync_copy(flag_v, ok_hbm.at[pl.ds(wid * cpw * 16, cpw * 16)])

    return sample


def _slow_path(x):
    colmask = _colmask_call(x)
    dest = _dest_call(colmask)
    out = _make_scatter()(x.reshape(ROWS, F),
                          dest.reshape(ROWS // _CHUNK, _CHUNK))
    return out.reshape(B, L, F)


def kernel(inputs):
    copied = _tc_copy(inputs)
    flags = _make_sc_sample()(inputs.reshape(ROWS * F))
    ok = jnp.all(jnp.any(flags.reshape(L, 16) != 0, axis=1))
    return lax.cond(ok, lambda c, x: c, lambda c, x: _slow_path(x),
                    copied, inputs)


# speedup vs baseline: 2.4074x; 1.7838x over previous
import functools

import jax
import jax.numpy as jnp
from jax import lax
from jax.experimental import pallas as pl
from jax.experimental.pallas import tpu as pltpu
from jax.experimental.pallas import tpu_sc as plsc

B, L, F = 16, 2048, 512
ROWS = B * L

_LBLK = 128


def _mask_body(x_ref, o_ref):
    nz = (x_ref[...] != 0.0).astype(jnp.float32)
    s = jnp.sum(jnp.sum(nz, axis=2), axis=0, keepdims=True)
    o_ref[...] = (s > 0.0).astype(jnp.int32)


_colmask_call = pl.pallas_call(
    _mask_body,
    grid=(L // _LBLK,),
    in_specs=[pl.BlockSpec((B, _LBLK, F), lambda i: (0, i, 0))],
    out_specs=pl.BlockSpec((1, _LBLK), lambda i: (0, i)),
    out_shape=jax.ShapeDtypeStruct((1, L), jnp.int32),
)


def _dest_body(m_ref, o_ref):
    kept = m_ref[...]
    r = lax.broadcasted_iota(jnp.int32, (L, L), 0)
    c = lax.broadcasted_iota(jnp.int32, (L, L), 1)
    tri = (r <= c).astype(jnp.bfloat16)
    incl = lax.dot_general(
        kept.astype(jnp.bfloat16), tri,
        (((1,), (0,)), ((), ())),
        preferred_element_type=jnp.float32,
    ).astype(jnp.int32)
    total = jnp.sum(kept)
    pe = incl - kept
    col = lax.broadcasted_iota(jnp.int32, (1, L), 1)
    dest = jnp.where(kept > 0, pe, total + col - pe)
    row = lax.broadcasted_iota(jnp.int32, (B, L), 0)
    o_ref[...] = dest + row * L


_dest_call = pl.pallas_call(
    _dest_body,
    out_shape=jax.ShapeDtypeStruct((B, L), jnp.int32),
)


_CHUNK = 64


@functools.cache
def _make_scatter():
    info = plsc.get_sparse_core_info()
    nc, ns = info.num_cores, info.num_subcores
    nw = nc * ns
    rpw = ROWS // nw
    nchunks = rpw // _CHUNK
    mesh = plsc.VectorSubcoreMesh(core_axis_name="c", subcore_axis_name="s")

    @functools.partial(
        pl.kernel,
        mesh=mesh,
        out_type=jax.ShapeDtypeStruct((ROWS, F), jnp.float32),
        scratch_types=[
            pltpu.VMEM((nchunks, _CHUNK), jnp.int32),
            pltpu.VMEM((_CHUNK, F), jnp.float32),
            pltpu.VMEM((_CHUNK, F), jnp.float32),
            pltpu.SemaphoreType.DMA,
            pltpu.SemaphoreType.DMA,
        ],
    )
    def scatter(rows_hbm, idx_hbm, out_hbm, idx_v, rows_a, rows_b, sem_a,
                sem_b):
        wid = lax.axis_index("s") * nc + lax.axis_index("c")
        base = wid * rpw
        pltpu.sync_copy(idx_hbm.at[pl.ds(wid * nchunks, nchunks)], idx_v)

        bufs = (rows_a, rows_b)
        sems = (sem_a, sem_b)
        pending = [None, None]
        for j in range(nchunks):
            b = j & 1
            if pending[b] is not None:
                pending[b].wait()
            pltpu.sync_copy(rows_hbm.at[pl.ds(base + j * _CHUNK, _CHUNK)],
                            bufs[b])
            pending[b] = pltpu.async_copy(bufs[b], out_hbm.at[idx_v.at[j]],
                                          sems[b])
        pending[0].wait()
        pending[1].wait()

    return scatter


def _tc_copy_body(x_ref, o_ref):
    o_ref[...] = x_ref[...]


_tc_copy = pl.pallas_call(
    _tc_copy_body,
    grid=(16,),
    in_specs=[pl.BlockSpec((B, L // 16, F), lambda i: (0, i, 0))],
    out_specs=pl.BlockSpec((B, L // 16, F), lambda i: (0, i, 0)),
    out_shape=jax.ShapeDtypeStruct((B, L, F), jnp.float32),
)


@functools.cache
def _make_sc_sample():
    info = plsc.get_sparse_core_info()
    nc, ns = info.num_cores, info.num_subcores
    nw = nc * ns
    cpw = L // nw
    mesh = plsc.VectorSubcoreMesh(core_axis_name="c", subcore_axis_name="s")

    @functools.partial(
        pl.kernel,
        mesh=mesh,
        out_type=jax.ShapeDtypeStruct((nw * cpw * 16,), jnp.float32),
        scratch_types=[
            pltpu.VMEM((cpw * F,), jnp.float32),
            pltpu.VMEM((cpw * 16,), jnp.float32),
            pltpu.SemaphoreType.DMA,
        ],
    )
    def sample(rows_hbm, ok_hbm, smp_v, flag_v, sem):
        wid = lax.axis_index("s") * nc + lax.axis_index("c")
        handles = [
            pltpu.async_copy(rows_hbm.at[wid * cpw + c],
                             smp_v.at[pl.ds(c * F, F)], sem)
            for c in range(cpw)
        ]
        for h in handles:
            h.wait()
        for c in range(cpw):
            acc = jnp.zeros((16,), jnp.float32)
            for k in range(8):
                acc = acc + jnp.abs(smp_v[pl.ds(c * F + 16 * k, 16)])
            flag_v[pl.ds(c * 16, 16)] = acc
        pltpu.sync_copy(flag_v, ok_hbm.at[pl.ds(wid * cpw * 16, cpw * 16)])

    return sample


def _slow_path(x):
    colmask = _colmask_call(x)
    dest = _dest_call(colmask)
    out = _make_scatter()(x.reshape(ROWS, F),
                          dest.reshape(ROWS // _CHUNK, _CHUNK))
    return out.reshape(B, L, F)


def kernel(inputs):
    copied = _tc_copy(inputs)
    flags = _make_sc_sample()(inputs.reshape(ROWS, F))
    ok = jnp.all(jnp.any(flags.reshape(L, 16) != 0, axis=1))
    return lax.cond(ok, lambda c, x: c, lambda c, x: _slow_path(x),
                    copied, inputs)


# speedup vs baseline: 2.4712x; 1.0265x over previous
import functools

import jax
import jax.numpy as jnp
from jax import lax
from jax.experimental import pallas as pl
from jax.experimental.pallas import tpu as pltpu
from jax.experimental.pallas import tpu_sc as plsc

B, L, F = 16, 2048, 512
ROWS = B * L

_LBLK = 128


def _mask_body(x_ref, o_ref):
    nz = (x_ref[...] != 0.0).astype(jnp.float32)
    s = jnp.sum(jnp.sum(nz, axis=2), axis=0, keepdims=True)
    o_ref[...] = (s > 0.0).astype(jnp.int32)


_colmask_call = pl.pallas_call(
    _mask_body,
    grid=(L // _LBLK,),
    in_specs=[pl.BlockSpec((B, _LBLK, F), lambda i: (0, i, 0))],
    out_specs=pl.BlockSpec((1, _LBLK), lambda i: (0, i)),
    out_shape=jax.ShapeDtypeStruct((1, L), jnp.int32),
)


def _dest_body(m_ref, o_ref):
    kept = m_ref[...]
    r = lax.broadcasted_iota(jnp.int32, (L, L), 0)
    c = lax.broadcasted_iota(jnp.int32, (L, L), 1)
    tri = (r <= c).astype(jnp.bfloat16)
    incl = lax.dot_general(
        kept.astype(jnp.bfloat16), tri,
        (((1,), (0,)), ((), ())),
        preferred_element_type=jnp.float32,
    ).astype(jnp.int32)
    total = jnp.sum(kept)
    pe = incl - kept
    col = lax.broadcasted_iota(jnp.int32, (1, L), 1)
    dest = jnp.where(kept > 0, pe, total + col - pe)
    row = lax.broadcasted_iota(jnp.int32, (B, L), 0)
    o_ref[...] = dest + row * L


_dest_call = pl.pallas_call(
    _dest_body,
    out_shape=jax.ShapeDtypeStruct((B, L), jnp.int32),
)


_CHUNK = 64


@functools.cache
def _make_scatter():
    info = plsc.get_sparse_core_info()
    nc, ns = info.num_cores, info.num_subcores
    nw = nc * ns
    rpw = ROWS // nw
    nchunks = rpw // _CHUNK
    mesh = plsc.VectorSubcoreMesh(core_axis_name="c", subcore_axis_name="s")

    @functools.partial(
        pl.kernel,
        mesh=mesh,
        out_type=jax.ShapeDtypeStruct((ROWS, F), jnp.float32),
        scratch_types=[
            pltpu.VMEM((nchunks, _CHUNK), jnp.int32),
            pltpu.VMEM((_CHUNK, F), jnp.float32),
            pltpu.VMEM((_CHUNK, F), jnp.float32),
            pltpu.SemaphoreType.DMA,
            pltpu.SemaphoreType.DMA,
        ],
    )
    def scatter(rows_hbm, idx_hbm, out_hbm, idx_v, rows_a, rows_b, sem_a,
                sem_b):
        wid = lax.axis_index("s") * nc + lax.axis_index("c")
        base = wid * rpw
        pltpu.sync_copy(idx_hbm.at[pl.ds(wid * nchunks, nchunks)], idx_v)

        bufs = (rows_a, rows_b)
        sems = (sem_a, sem_b)
        pending = [None, None]
        for j in range(nchunks):
            b = j & 1
            if pending[b] is not None:
                pending[b].wait()
            pltpu.sync_copy(rows_hbm.at[pl.ds(base + j * _CHUNK, _CHUNK)],
                            bufs[b])
            pending[b] = pltpu.async_copy(bufs[b], out_hbm.at[idx_v.at[j]],
                                          sems[b])
        pending[0].wait()
        pending[1].wait()

    return scatter


def _tc_copy_body(x_ref, o_ref):
    o_ref[...] = x_ref[...]


_TCBLK = 8
_tc_copy = pl.pallas_call(
    _tc_copy_body,
    grid=(_TCBLK,),
    in_specs=[pl.BlockSpec((B, L // _TCBLK, F), lambda i: (0, i, 0))],
    out_specs=pl.BlockSpec((B, L // _TCBLK, F), lambda i: (0, i, 0)),
    out_shape=jax.ShapeDtypeStruct((B, L, F), jnp.float32),
)


@functools.cache
def _make_sc_sample():
    info = plsc.get_sparse_core_info()
    nc, ns = info.num_cores, info.num_subcores
    nw = nc * ns
    cpw = L // nw
    mesh = plsc.VectorSubcoreMesh(core_axis_name="c", subcore_axis_name="s")

    @functools.partial(
        pl.kernel,
        mesh=mesh,
        out_type=jax.ShapeDtypeStruct((nw * cpw * 16,), jnp.float32),
        scratch_types=[
            pltpu.VMEM((cpw * F,), jnp.float32),
            pltpu.VMEM((cpw * 16,), jnp.float32),
            pltpu.SemaphoreType.DMA,
        ],
    )
    def sample(rows_hbm, ok_hbm, smp_v, flag_v, sem):
        wid = lax.axis_index("s") * nc + lax.axis_index("c")
        handles = [
            pltpu.async_copy(rows_hbm.at[wid * cpw + c],
                             smp_v.at[pl.ds(c * F, F)], sem)
            for c in range(cpw)
        ]
        for h in handles:
            h.wait()
        for c in range(cpw):
            acc = jnp.zeros((16,), jnp.float32)
            for k in range(8):
                acc = acc + jnp.abs(smp_v[pl.ds(c * F + 16 * k, 16)])
            flag_v[pl.ds(c * 16, 16)] = acc
        pltpu.sync_copy(flag_v, ok_hbm.at[pl.ds(wid * cpw * 16, cpw * 16)])

    return sample


def _slow_path(x):
    colmask = _colmask_call(x)
    dest = _dest_call(colmask)
    out = _make_scatter()(x.reshape(ROWS, F),
                          dest.reshape(ROWS // _CHUNK, _CHUNK))
    return out.reshape(B, L, F)


def kernel(inputs):
    copied = _tc_copy(inputs)
    flags = _make_sc_sample()(inputs.reshape(ROWS, F))
    ok = jnp.all(jnp.any(flags.reshape(L, 16) != 0, axis=1))
    return lax.cond(ok, lambda c, x: c, lambda c, x: _slow_path(x),
                    copied, inputs)


# speedup vs baseline: 2.5223x; 1.0207x over previous
import functools

import jax
import jax.numpy as jnp
from jax import lax
from jax.experimental import pallas as pl
from jax.experimental.pallas import tpu as pltpu
from jax.experimental.pallas import tpu_sc as plsc

B, L, F = 16, 2048, 512
ROWS = B * L

_LBLK = 128


def _mask_body(x_ref, o_ref):
    nz = (x_ref[...] != 0.0).astype(jnp.float32)
    s = jnp.sum(jnp.sum(nz, axis=2), axis=0, keepdims=True)
    o_ref[...] = (s > 0.0).astype(jnp.int32)


_colmask_call = pl.pallas_call(
    _mask_body,
    grid=(L // _LBLK,),
    in_specs=[pl.BlockSpec((B, _LBLK, F), lambda i: (0, i, 0))],
    out_specs=pl.BlockSpec((1, _LBLK), lambda i: (0, i)),
    out_shape=jax.ShapeDtypeStruct((1, L), jnp.int32),
)


def _dest_body(m_ref, o_ref):
    kept = m_ref[...]
    r = lax.broadcasted_iota(jnp.int32, (L, L), 0)
    c = lax.broadcasted_iota(jnp.int32, (L, L), 1)
    tri = (r <= c).astype(jnp.bfloat16)
    incl = lax.dot_general(
        kept.astype(jnp.bfloat16), tri,
        (((1,), (0,)), ((), ())),
        preferred_element_type=jnp.float32,
    ).astype(jnp.int32)
    total = jnp.sum(kept)
    pe = incl - kept
    col = lax.broadcasted_iota(jnp.int32, (1, L), 1)
    dest = jnp.where(kept > 0, pe, total + col - pe)
    row = lax.broadcasted_iota(jnp.int32, (B, L), 0)
    o_ref[...] = dest + row * L


_dest_call = pl.pallas_call(
    _dest_body,
    out_shape=jax.ShapeDtypeStruct((B, L), jnp.int32),
)


_CHUNK = 64


@functools.cache
def _make_scatter():
    info = plsc.get_sparse_core_info()
    nc, ns = info.num_cores, info.num_subcores
    nw = nc * ns
    rpw = ROWS // nw
    nchunks = rpw // _CHUNK
    mesh = plsc.VectorSubcoreMesh(core_axis_name="c", subcore_axis_name="s")

    @functools.partial(
        pl.kernel,
        mesh=mesh,
        out_type=jax.ShapeDtypeStruct((ROWS, F), jnp.float32),
        scratch_types=[
            pltpu.VMEM((nchunks, _CHUNK), jnp.int32),
            pltpu.VMEM((_CHUNK, F), jnp.float32),
            pltpu.VMEM((_CHUNK, F), jnp.float32),
            pltpu.SemaphoreType.DMA,
            pltpu.SemaphoreType.DMA,
        ],
    )
    def scatter(rows_hbm, idx_hbm, out_hbm, idx_v, rows_a, rows_b, sem_a,
                sem_b):
        wid = lax.axis_index("s") * nc + lax.axis_index("c")
        base = wid * rpw
        pltpu.sync_copy(idx_hbm.at[pl.ds(wid * nchunks, nchunks)], idx_v)

        bufs = (rows_a, rows_b)
        sems = (sem_a, sem_b)
        pending = [None, None]
        for j in range(nchunks):
            b = j & 1
            if pending[b] is not None:
                pending[b].wait()
            pltpu.sync_copy(rows_hbm.at[pl.ds(base + j * _CHUNK, _CHUNK)],
                            bufs[b])
            pending[b] = pltpu.async_copy(bufs[b], out_hbm.at[idx_v.at[j]],
                                          sems[b])
        pending[0].wait()
        pending[1].wait()

    return scatter


def _copy_mask_body(x_ref, o_ref, m_ref):
    x = x_ref[...]
    o_ref[...] = x
    bits = lax.bitcast_convert_type(x, jnp.int32) & jnp.int32(0x7FFFFFFF)
    m = jnp.max(jnp.max(bits, axis=2), axis=0, keepdims=True)
    m_ref[...] = jnp.minimum(m, 1)


_TCBLK = 8
_tc_copy = pl.pallas_call(
    _copy_mask_body,
    grid=(_TCBLK,),
    in_specs=[pl.BlockSpec((B, L // _TCBLK, F), lambda i: (0, i, 0))],
    out_specs=[
        pl.BlockSpec((B, L // _TCBLK, F), lambda i: (0, i, 0)),
        pl.BlockSpec((1, L // _TCBLK), lambda i: (0, i)),
    ],
    out_shape=[
        jax.ShapeDtypeStruct((B, L, F), jnp.float32),
        jax.ShapeDtypeStruct((1, L), jnp.int32),
    ],
)


@functools.cache
def _make_sc_sample():
    info = plsc.get_sparse_core_info()
    nc, ns = info.num_cores, info.num_subcores
    nw = nc * ns
    cpw = L // nw
    mesh = plsc.VectorSubcoreMesh(core_axis_name="c", subcore_axis_name="s")

    @functools.partial(
        pl.kernel,
        mesh=mesh,
        out_type=jax.ShapeDtypeStruct((nw * cpw * 16,), jnp.float32),
        scratch_types=[
            pltpu.VMEM((cpw * F,), jnp.float32),
            pltpu.VMEM((cpw * 16,), jnp.float32),
            pltpu.SemaphoreType.DMA,
        ],
    )
    def sample(rows_hbm, ok_hbm, smp_v, flag_v, sem):
        wid = lax.axis_index("s") * nc + lax.axis_index("c")
        handles = [
            pltpu.async_copy(rows_hbm.at[wid * cpw + c],
                             smp_v.at[pl.ds(c * F, F)], sem)
            for c in range(cpw)
        ]
        for h in handles:
            h.wait()
        for c in range(cpw):
            acc = jnp.zeros((16,), jnp.float32)
            for k in range(8):
                acc = acc + jnp.abs(smp_v[pl.ds(c * F + 16 * k, 16)])
            flag_v[pl.ds(c * 16, 16)] = acc
        pltpu.sync_copy(flag_v, ok_hbm.at[pl.ds(wid * cpw * 16, cpw * 16)])

    return sample


def _slow_path(copied, x, colmask):
    dest = _dest_call(colmask)
    out = _make_scatter()(x.reshape(ROWS, F),
                          dest.reshape(ROWS // _CHUNK, _CHUNK))
    return out.reshape(B, L, F)


def kernel(inputs):
    copied, colmask = _tc_copy(inputs)
    ok = jnp.min(colmask) > 0
    return lax.cond(ok, lambda c, x, m: c, _slow_path,
                    copied, inputs, colmask)


# speedup vs baseline: 2.5448x; 1.0089x over previous
import functools

import jax
import jax.numpy as jnp
from jax import lax
from jax.experimental import pallas as pl
from jax.experimental.pallas import tpu as pltpu
from jax.experimental.pallas import tpu_sc as plsc

B, L, F = 16, 2048, 512
ROWS = B * L

_LBLK = 128


def _mask_body(x_ref, o_ref):
    nz = (x_ref[...] != 0.0).astype(jnp.float32)
    s = jnp.sum(jnp.sum(nz, axis=2), axis=0, keepdims=True)
    o_ref[...] = (s > 0.0).astype(jnp.int32)


_colmask_call = pl.pallas_call(
    _mask_body,
    grid=(L // _LBLK,),
    in_specs=[pl.BlockSpec((B, _LBLK, F), lambda i: (0, i, 0))],
    out_specs=pl.BlockSpec((1, _LBLK), lambda i: (0, i)),
    out_shape=jax.ShapeDtypeStruct((1, L), jnp.int32),
)


def _dest_body(m_ref, o_ref):
    kept = m_ref[...]
    r = lax.broadcasted_iota(jnp.int32, (L, L), 0)
    c = lax.broadcasted_iota(jnp.int32, (L, L), 1)
    tri = (r <= c).astype(jnp.bfloat16)
    incl = lax.dot_general(
        kept.astype(jnp.bfloat16), tri,
        (((1,), (0,)), ((), ())),
        preferred_element_type=jnp.float32,
    ).astype(jnp.int32)
    total = jnp.sum(kept)
    pe = incl - kept
    col = lax.broadcasted_iota(jnp.int32, (1, L), 1)
    dest = jnp.where(kept > 0, pe, total + col - pe)
    row = lax.broadcasted_iota(jnp.int32, (B, L), 0)
    o_ref[...] = dest + row * L


_dest_call = pl.pallas_call(
    _dest_body,
    out_shape=jax.ShapeDtypeStruct((B, L), jnp.int32),
)


_CHUNK = 64


@functools.cache
def _make_scatter():
    info = plsc.get_sparse_core_info()
    nc, ns = info.num_cores, info.num_subcores
    nw = nc * ns
    rpw = ROWS // nw
    nchunks = rpw // _CHUNK
    mesh = plsc.VectorSubcoreMesh(core_axis_name="c", subcore_axis_name="s")

    @functools.partial(
        pl.kernel,
        mesh=mesh,
        out_type=jax.ShapeDtypeStruct((ROWS, F), jnp.float32),
        scratch_types=[
            pltpu.VMEM((nchunks, _CHUNK), jnp.int32),
            pltpu.VMEM((_CHUNK, F), jnp.float32),
            pltpu.VMEM((_CHUNK, F), jnp.float32),
            pltpu.SemaphoreType.DMA,
            pltpu.SemaphoreType.DMA,
        ],
    )
    def scatter(rows_hbm, idx_hbm, out_hbm, idx_v, rows_a, rows_b, sem_a,
                sem_b):
        wid = lax.axis_index("s") * nc + lax.axis_index("c")
        base = wid * rpw
        pltpu.sync_copy(idx_hbm.at[pl.ds(wid * nchunks, nchunks)], idx_v)

        bufs = (rows_a, rows_b)
        sems = (sem_a, sem_b)
        pending = [None, None]
        for j in range(nchunks):
            b = j & 1
            if pending[b] is not None:
                pending[b].wait()
            pltpu.sync_copy(rows_hbm.at[pl.ds(base + j * _CHUNK, _CHUNK)],
                            bufs[b])
            pending[b] = pltpu.async_copy(bufs[b], out_hbm.at[idx_v.at[j]],
                                          sems[b])
        pending[0].wait()
        pending[1].wait()

    return scatter


def _copy_mask_body(x_ref, o_ref, m_ref):
    x = x_ref[...]
    o_ref[...] = x
    bits = lax.bitcast_convert_type(x, jnp.int32) & jnp.int32(0x7FFFFFFF)
    m = jnp.max(jnp.max(bits, axis=2), axis=0, keepdims=True)
    m_ref[...] = jnp.minimum(m, 1)


_TCBLK = 8
_tc_copy = pl.pallas_call(
    _copy_mask_body,
    grid=(_TCBLK,),
    in_specs=[pl.BlockSpec((B, L // _TCBLK, F), lambda i: (0, i, 0))],
    out_specs=[
        pl.BlockSpec((B, L // _TCBLK, F), lambda i: (0, i, 0)),
        pl.BlockSpec((1, L // _TCBLK), lambda i: (0, i)),
    ],
    out_shape=[
        jax.ShapeDtypeStruct((B, L, F), jnp.float32),
        jax.ShapeDtypeStruct((1, L), jnp.int32),
    ],
)


@functools.cache
def _make_sc_sample():
    info = plsc.get_sparse_core_info()
    nc, ns = info.num_cores, info.num_subcores
    nw = nc * ns
    cpw = L // nw
    mesh = plsc.VectorSubcoreMesh(core_axis_name="c", subcore_axis_name="s")

    @functools.partial(
        pl.kernel,
        mesh=mesh,
        out_type=jax.ShapeDtypeStruct((nw * cpw * 16,), jnp.float32),
        scratch_types=[
            pltpu.VMEM((cpw * F,), jnp.float32),
            pltpu.VMEM((cpw * 16,), jnp.float32),
            pltpu.SemaphoreType.DMA,
        ],
    )
    def sample(rows_hbm, ok_hbm, smp_v, flag_v, sem):
        wid = lax.axis_index("s") * nc + lax.axis_index("c")
        handles = [
            pltpu.async_copy(rows_hbm.at[wid * cpw + c],
                             smp_v.at[pl.ds(c * F, F)], sem)
            for c in range(cpw)
        ]
        for h in handles:
            h.wait()
        for c in range(cpw):
            acc = jnp.zeros((16,), jnp.float32)
            for k in range(8):
                acc = acc + jnp.abs(smp_v[pl.ds(c * F + 16 * k, 16)])
            flag_v[pl.ds(c * 16, 16)] = acc
        pltpu.sync_copy(flag_v, ok_hbm.at[pl.ds(wid * cpw * 16, cpw * 16)])

    return sample


def _slow_path(copied, colmask):
    dest = _dest_call(colmask)
    out = _make_scatter()(copied.reshape(ROWS, F),
                          dest.reshape(ROWS // _CHUNK, _CHUNK))
    return out.reshape(B, L, F)


def kernel(inputs):
    copied, colmask = _tc_copy(inputs)
    ok = jnp.min(colmask) > 0
    return lax.cond(ok, lambda c, m: c, _slow_path, copied, colmask)
